# baseline (speedup 1.0000x reference)
import functools

import jax
import jax.numpy as jnp
from jax import lax
from jax.experimental import pallas as pl
from jax.experimental.pallas import tpu as pltpu
from jax.experimental.pallas import tpu_sc as plsc

NUM_EMB = 1000000
DIM = 64
BATCH = 16384

NC = 2
NS = 16
NW = NC * NS
B_PER_W = BATCH // NW
CHUNK = 128
NCHUNK = B_PER_W // CHUNK


def _body(w_hbm, x_hbm, o_hbm, idx_v, rows_v, gsem, ssem):
    wid = lax.axis_index("s") * NC + lax.axis_index("c")
    pltpu.sync_copy(x_hbm.at[wid], idx_v)
    gathers = [
        pltpu.async_copy(
            w_hbm.at[idx_v.at[j]], rows_v.at[pl.ds(j * CHUNK, CHUNK)], gsem
        )
        for j in range(NCHUNK)
    ]
    scatters = []
    for j in range(NCHUNK):
        gathers[j].wait()
        scatters.append(
            pltpu.async_copy(
                rows_v.at[pl.ds(j * CHUNK, CHUNK)],
                o_hbm.at[pl.ds(wid * B_PER_W + j * CHUNK, CHUNK)],
                ssem,
            )
        )
    for s in scatters:
        s.wait()


@functools.partial(jax.jit, static_argnames=())
def kernel(x, weight):
    xr = x.astype(jnp.int32).reshape(NW, NCHUNK, CHUNK)
    mesh = plsc.VectorSubcoreMesh(core_axis_name="c", subcore_axis_name="s")
    fn = pl.kernel(
        _body,
        out_type=jax.ShapeDtypeStruct((BATCH, DIM), jnp.float32),
        mesh=mesh,
        scratch_types=[
            pltpu.VMEM((NCHUNK, CHUNK), jnp.int32),
            pltpu.VMEM((B_PER_W, DIM), jnp.float32),
            pltpu.SemaphoreType.DMA,
            pltpu.SemaphoreType.DMA,
        ],
        compiler_params=pltpu.CompilerParams(use_tc_tiling_on_sc=False),
    )
    return fn(weight, xr)


# speedup vs baseline: 1.7209x; 1.7209x over previous
import functools

import jax
import jax.numpy as jnp
from jax import lax
from jax.experimental import pallas as pl
from jax.experimental.pallas import tpu as pltpu
from jax.experimental.pallas import tpu_sc as plsc

NUM_EMB = 1000000
DIM = 64
BATCH = 16384

NC = 2
NS = 16
NW = NC * NS
B_PER_W = BATCH // NW


def _body(w_hbm, x_hbm, o_hbm, idx_v, rows_v, gsem):
    wid = lax.axis_index("s") * NC + lax.axis_index("c")
    base = wid * B_PER_W
    pltpu.sync_copy(x_hbm.at[pl.ds(base, B_PER_W)], idx_v)

    def enqueue16(k, carry):
        v = idx_v[pl.ds(k * 16, 16)]
        for l in range(16):
            pltpu.async_copy(
                w_hbm.at[pl.ds(v[l], 1)],
                rows_v.at[pl.ds(k * 16 + l, 1)],
                gsem,
            )
        return carry

    lax.fori_loop(0, B_PER_W // 16, enqueue16, 0)
    pltpu.make_async_copy(w_hbm.at[pl.ds(0, B_PER_W)], rows_v, gsem).wait()
    pltpu.sync_copy(rows_v, o_hbm.at[pl.ds(base, B_PER_W)])


@jax.jit
def kernel(x, weight):
    xi = x.astype(jnp.int32)
    mesh = plsc.VectorSubcoreMesh(core_axis_name="c", subcore_axis_name="s")
    fn = pl.kernel(
        _body,
        out_type=jax.ShapeDtypeStruct((BATCH, DIM), jnp.float32),
        mesh=mesh,
        scratch_types=[
            pltpu.VMEM((B_PER_W,), jnp.int32),
            pltpu.VMEM((B_PER_W, DIM), jnp.float32),
            pltpu.SemaphoreType.DMA,
        ],
    )
    return fn(weight, xi)


# speedup vs baseline: 5.1330x; 2.9827x over previous
import jax
import jax.numpy as jnp
from jax import lax
from jax.experimental import pallas as pl
from jax.experimental.pallas import tpu as pltpu
from jax.experimental.pallas import tpu_sc as plsc

NUM_EMB = 1000000
DIM = 64
BATCH = 16384

NC = 2
NS = 16
NW = NC * NS
B_PER_W = BATCH // NW
NB = 8


def _row_scalar(idxs_v, n):
    lane = jnp.broadcast_to(n, (16,)).astype(jnp.int32)
    return jnp.max(plsc.load_gather(idxs_v, [lane]))


def _body(wt, x_hbm, o_hbm, idxs_v, tb_v, obuf_v, gsem, ssem):
    wid = lax.axis_index("s") * NC + lax.axis_index("c")
    base = wid * B_PER_W
    pltpu.sync_copy(x_hbm.at[pl.ds(base, B_PER_W)], idxs_v)

    def fetch(n):
        r = _row_scalar(idxs_v, n)
        off = pl.multiple_of((r >> 7) * 128, 128)
        b = (n & (NB - 1)) * 8
        for i in range(8):
            pltpu.async_copy(
                wt.at[pl.ds(8 * i, 8), pl.ds(off, 128)], tb_v.at[b + i], gsem
            )

    def wait_row(n, r):
        off = pl.multiple_of((r >> 7) * 128, 128)
        b = (n & (NB - 1)) * 8
        for i in range(8):
            pltpu.make_async_copy(
                wt.at[pl.ds(8 * i, 8), pl.ds(off, 128)], tb_v.at[b + i], gsem
            ).wait()

    iota = lax.iota(jnp.int32, 16)

    def process(n):
        r = _row_scalar(idxs_v, n)
        wait_row(n, r)
        b = (n & (NB - 1)) * 8
        rl = jnp.broadcast_to(r & 127, (16,)).astype(jnp.int32)
        m = n >> 7
        lane = n & 127
        pobase = m * 1024 + lane
        for k in range(4):
            ch = 2 * k + (iota >> 3)
            cl = iota & 7
            vals = plsc.load_gather(tb_v, [b + ch, cl, rl])
            pos = ch * (8 * 1024) + cl * 128 + pobase
            plsc.store_scatter(obuf_v, [pos], vals)

    def prologue(n, c):
        fetch(n)
        return c

    def steady(n, c):
        process(n)
        fetch(n + NB)
        return c

    def tail(n, c):
        process(n)
        return c

    lax.fori_loop(0, NB, prologue, 0)
    lax.fori_loop(0, B_PER_W - NB, steady, 0)
    lax.fori_loop(B_PER_W - NB, B_PER_W, tail, 0)

    def sloop(s, c):
        ch = s >> 2
        m = s & 3
        off = ch * (BATCH * 8) + (wid * 4 + m) * 1024
        pltpu.async_copy(
            obuf_v.at[pl.ds(s * 1024, 1024)], o_hbm.at[pl.ds(off, 1024)], ssem
        )
        return c

    lax.fori_loop(0, 32, sloop, 0)
    pltpu.make_async_copy(
        o_hbm.at[pl.ds(0, 32768)], o_hbm.at[pl.ds(0, 32768)], ssem
    ).wait()


@jax.jit
def kernel(x, weight):
    xi = x.astype(jnp.int32)
    wt = weight.T
    mesh = plsc.VectorSubcoreMesh(core_axis_name="c", subcore_axis_name="s")
    fn = pl.kernel(
        _body,
        out_type=jax.ShapeDtypeStruct((BATCH * DIM,), jnp.float32),
        mesh=mesh,
        scratch_types=[
            pltpu.VMEM((B_PER_W,), jnp.int32),
            pltpu.VMEM((NB * 8, 8, 128), jnp.float32),
            pltpu.VMEM((B_PER_W * DIM,), jnp.float32),
            pltpu.SemaphoreType.DMA,
            pltpu.SemaphoreType.DMA,
        ],
        compiler_params=pltpu.CompilerParams(
            use_tc_tiling_on_sc=True, needs_layout_passes=False
        ),
    )
    oflat = fn(wt, xi)
    return (
        oflat.reshape(DIM // 8, BATCH // 128, 8, 128)
        .transpose(1, 3, 0, 2)
        .reshape(BATCH, DIM)
    )


# speedup vs baseline: 5.1465x; 1.0026x over previous
import jax
import jax.numpy as jnp
from jax import lax
from jax.experimental import pallas as pl
from jax.experimental.pallas import tpu as pltpu
from jax.experimental.pallas import tpu_sc as plsc

NUM_EMB = 1000000
DIM = 64
BATCH = 16384

NC = 2
NS = 16
NW = NC * NS
B_PER_W = BATCH // NW
NB = 8


def _row_scalar(idxs_v, n):
    lane = jnp.broadcast_to(n, (16,)).astype(jnp.int32)
    return jnp.max(plsc.load_gather(idxs_v, [lane]))


def _body(wt, x_hbm, o_hbm, idxs_v, tb_v, obuf_v, gsem, ssem):
    wid = lax.axis_index("s") * NC + lax.axis_index("c")
    base = wid * B_PER_W
    pltpu.sync_copy(x_hbm.at[pl.ds(base, B_PER_W)], idxs_v)

    def fetch(n):
        r = _row_scalar(idxs_v, n)
        off = pl.multiple_of((r >> 7) * 128, 128)
        b = n & (NB - 1)
        pltpu.async_copy(wt.at[:, pl.ds(off, 128)], tb_v.at[b], gsem)

    iota = lax.iota(jnp.int32, 16)

    def process(n):
        r = _row_scalar(idxs_v, n)
        off = pl.multiple_of((r >> 7) * 128, 128)
        b = n & (NB - 1)
        pltpu.make_async_copy(wt.at[:, pl.ds(off, 128)], tb_v.at[b], gsem).wait()
        rl = jnp.broadcast_to(r & 127, (16,)).astype(jnp.int32)
        m = n >> 7
        lane = n & 127
        pobase = m * 1024 + lane
        for k in range(4):
            c = 16 * k + iota
            vals = plsc.load_gather(tb_v, [jnp.broadcast_to(b, (16,)), c, rl])
            pos = (c >> 3) * (8 * 1024) + (c & 7) * 128 + pobase
            plsc.store_scatter(obuf_v, [pos], vals)

    def prologue(n, c):
        fetch(n)
        return c

    def steady(n, c):
        process(n)
        fetch(n + NB)
        return c

    def tail(n, c):
        process(n)
        return c

    lax.fori_loop(0, NB, prologue, 0)
    lax.fori_loop(0, B_PER_W - NB, steady, 0)
    lax.fori_loop(B_PER_W - NB, B_PER_W, tail, 0)

    def sloop(s, c):
        ch = s >> 2
        m = s & 3
        off = ch * (BATCH * 8) + (wid * 4 + m) * 1024
        pltpu.async_copy(
            obuf_v.at[pl.ds(s * 1024, 1024)], o_hbm.at[pl.ds(off, 1024)], ssem
        )
        return c

    lax.fori_loop(0, 32, sloop, 0)
    pltpu.make_async_copy(
        o_hbm.at[pl.ds(0, 32768)], o_hbm.at[pl.ds(0, 32768)], ssem
    ).wait()


@jax.jit
def kernel(x, weight):
    xi = x.astype(jnp.int32)
    wt = weight.T
    mesh = plsc.VectorSubcoreMesh(core_axis_name="c", subcore_axis_name="s")
    fn = pl.kernel(
        _body,
        out_type=jax.ShapeDtypeStruct((BATCH * DIM,), jnp.float32),
        mesh=mesh,
        scratch_types=[
            pltpu.VMEM((B_PER_W,), jnp.int32),
            pltpu.VMEM((NB, DIM, 128), jnp.float32),
            pltpu.VMEM((B_PER_W * DIM,), jnp.float32),
            pltpu.SemaphoreType.DMA,
            pltpu.SemaphoreType.DMA,
        ],
        compiler_params=pltpu.CompilerParams(
            use_tc_tiling_on_sc=True, needs_layout_passes=False
        ),
    )
    oflat = fn(wt, xi)
    return (
        oflat.reshape(DIM // 8, BATCH // 128, 8, 128)
        .transpose(1, 3, 0, 2)
        .reshape(BATCH, DIM)
    )
